# initial kernel scaffold (unmeasured)
import jax
import jax.numpy as jnp
from jax import lax
from jax.experimental import pallas as pl
from jax.experimental.pallas import tpu as pltpu


def kernel(
    x,
):
    def body(*refs):
        pass

    out_shape = jax.ShapeDtypeStruct(..., jnp.float32)
    return pl.pallas_call(body, out_shape=out_shape)(...)


# baseline (device time: 88099 ns/iter reference)
import jax
import jax.numpy as jnp
from jax import lax
from jax.experimental import pallas as pl
from jax.experimental.pallas import tpu as pltpu

N_DEV = 4


def kernel(x):
    m, n = x.shape
    m_ch = m // N_DEV

    def body(x_ref, out_ref, rs_ref, ag_ref, rs_send, rs_recv, ag_send, ag_recv):
        my = lax.axis_index("i")
        left = lax.rem(my + (N_DEV - 1), N_DEV)
        right = lax.rem(my + 1, N_DEV)

        barrier_sem = pltpu.get_barrier_semaphore()
        for nbr in [left, right]:
            pl.semaphore_signal(
                barrier_sem,
                inc=1,
                device_id=(nbr,),
                device_id_type=pl.DeviceIdType.MESH,
            )
        pl.semaphore_wait(barrier_sem, 2)

        rs_ref[0] = x_ref[pl.ds(my * m_ch, m_ch), :].astype(jnp.bfloat16)
        for s in range(N_DEV - 1):
            rdma = pltpu.make_async_remote_copy(
                src_ref=rs_ref.at[s],
                dst_ref=rs_ref.at[s + 1],
                send_sem=rs_send.at[s],
                recv_sem=rs_recv.at[s],
                device_id=(right,),
                device_id_type=pl.DeviceIdType.MESH,
            )
            rdma.start()
            rdma.wait()
            idx = lax.rem(my - 1 - s + 2 * N_DEV, N_DEV)
            rs_ref[s + 1] = rs_ref[s + 1] + x_ref[
                pl.ds(idx * m_ch, m_ch), :
            ].astype(jnp.bfloat16)

        own = lax.rem(my + 1, N_DEV)
        out_ref[pl.ds(own * m_ch, m_ch), :] = rs_ref[N_DEV - 1].astype(
            jnp.float32
        )

        ag_ref[0] = rs_ref[N_DEV - 1]
        for t in range(N_DEV - 1):
            rdma = pltpu.make_async_remote_copy(
                src_ref=ag_ref.at[t],
                dst_ref=ag_ref.at[t + 1],
                send_sem=ag_send.at[t],
                recv_sem=ag_recv.at[t],
                device_id=(right,),
                device_id_type=pl.DeviceIdType.MESH,
            )
            rdma.start()
            rdma.wait()
            idx = lax.rem(my - t + 2 * N_DEV, N_DEV)
            out_ref[pl.ds(idx * m_ch, m_ch), :] = ag_ref[t + 1].astype(
                jnp.float32
            )

    return pl.pallas_call(
        body,
        out_shape=jax.ShapeDtypeStruct((m, n), jnp.float32),
        in_specs=[pl.BlockSpec(memory_space=pltpu.VMEM)],
        out_specs=pl.BlockSpec(memory_space=pltpu.VMEM),
        scratch_shapes=[
            pltpu.VMEM((N_DEV, m_ch, n), jnp.bfloat16),
            pltpu.VMEM((N_DEV, m_ch, n), jnp.bfloat16),
            pltpu.SemaphoreType.DMA((N_DEV - 1,)),
            pltpu.SemaphoreType.DMA((N_DEV - 1,)),
            pltpu.SemaphoreType.DMA((N_DEV - 1,)),
            pltpu.SemaphoreType.DMA((N_DEV - 1,)),
        ],
        compiler_params=pltpu.CompilerParams(collective_id=0),
    )(x)


# device time: 51284 ns/iter; 1.7179x vs baseline; 1.7179x over previous
import jax
import jax.numpy as jnp
from jax import lax
from jax.experimental import pallas as pl
from jax.experimental.pallas import tpu as pltpu

N_DEV = 4


def kernel(x):
    m, n = x.shape
    mc = m // N_DEV
    hc = n // 2

    def body(
        x_ref,
        out_ref,
        accA,
        accB,
        rA1,
        rB1,
        rA2,
        rB2,
        send_sems,
        recv_sems,
    ):
        d = lax.axis_index("i")
        p1 = d ^ 1
        p2 = 3 - d

        barrier_sem = pltpu.get_barrier_semaphore()
        for nbr in [p1, p2]:
            pl.semaphore_signal(
                barrier_sem,
                inc=1,
                device_id=(nbr,),
                device_id_type=pl.DeviceIdType.MESH,
            )
        pl.semaphore_wait(barrier_sem, 2)

        ownA = lax.rem(d + 3, N_DEV)
        keepA = jnp.where(ownA >= 2, 2, 0)
        sendA1 = 2 - keepA
        sendA2 = lax.rem(6 - d, N_DEV)
        keepB = jnp.where(d >= 2, 2, 0)
        sendB1 = 2 - keepB
        sendB2 = p1

        for c in range(N_DEV):
            accA[c] = x_ref[pl.ds(c * mc, mc), 0:hc].astype(jnp.bfloat16)
            accB[c] = x_ref[pl.ds(c * mc, mc), hc:n].astype(jnp.bfloat16)

        def exchange(src, dst, sem_idx, partner):
            return pltpu.make_async_remote_copy(
                src_ref=src,
                dst_ref=dst,
                send_sem=send_sems.at[sem_idx],
                recv_sem=recv_sems.at[sem_idx],
                device_id=(partner,),
                device_id_type=pl.DeviceIdType.MESH,
            )

        a1 = exchange(accA.at[pl.ds(sendA1, 2)], rA1, 0, p1)
        b1 = exchange(accB.at[pl.ds(sendB1, 2)], rB1, 1, p2)
        a1.start()
        b1.start()
        a1.wait()
        accA[pl.ds(keepA, 2)] = accA[pl.ds(keepA, 2)] + rA1[...]
        b1.wait()
        accB[pl.ds(keepB, 2)] = accB[pl.ds(keepB, 2)] + rB1[...]

        a2 = exchange(accA.at[pl.ds(sendA2, 1)], rA2, 2, p2)
        b2 = exchange(accB.at[pl.ds(sendB2, 1)], rB2, 3, p1)
        a2.start()
        b2.start()
        a2.wait()
        accA[pl.ds(ownA, 1)] = accA[pl.ds(ownA, 1)] + rA2[...]
        out_ref[pl.ds(ownA * mc, mc), 0:hc] = accA[pl.ds(ownA, 1)].reshape(
            mc, hc
        ).astype(jnp.float32)
        b2.wait()
        accB[pl.ds(d, 1)] = accB[pl.ds(d, 1)] + rB2[...]
        out_ref[pl.ds(d * mc, mc), hc:n] = accB[pl.ds(d, 1)].reshape(
            mc, hc
        ).astype(jnp.float32)

        g1a = exchange(accA.at[pl.ds(ownA, 1)], accA.at[pl.ds(ownA, 1)], 4, p2)
        g1b = exchange(accB.at[pl.ds(d, 1)], accB.at[pl.ds(d, 1)], 5, p1)
        g1a.start()
        g1b.start()
        g1a.wait()
        out_ref[pl.ds(sendA2 * mc, mc), 0:hc] = accA[
            pl.ds(sendA2, 1)
        ].reshape(mc, hc).astype(jnp.float32)
        g1b.wait()
        out_ref[pl.ds(p1 * mc, mc), hc:n] = accB[pl.ds(p1, 1)].reshape(
            mc, hc
        ).astype(jnp.float32)

        g2a = exchange(
            accA.at[pl.ds(keepA, 2)], accA.at[pl.ds(keepA, 2)], 6, p1
        )
        g2b = exchange(
            accB.at[pl.ds(keepB, 2)], accB.at[pl.ds(keepB, 2)], 7, p2
        )
        g2a.start()
        g2b.start()
        g2a.wait()
        out_ref[pl.ds((2 - keepA) * mc, 2 * mc), 0:hc] = accA[
            pl.ds(2 - keepA, 2)
        ].reshape(2 * mc, hc).astype(jnp.float32)
        g2b.wait()
        out_ref[pl.ds((2 - keepB) * mc, 2 * mc), hc:n] = accB[
            pl.ds(2 - keepB, 2)
        ].reshape(2 * mc, hc).astype(jnp.float32)

    return pl.pallas_call(
        body,
        out_shape=jax.ShapeDtypeStruct((m, n), jnp.float32),
        in_specs=[pl.BlockSpec(memory_space=pltpu.VMEM)],
        out_specs=pl.BlockSpec(memory_space=pltpu.VMEM),
        scratch_shapes=[
            pltpu.VMEM((N_DEV, mc, hc), jnp.bfloat16),
            pltpu.VMEM((N_DEV, mc, hc), jnp.bfloat16),
            pltpu.VMEM((2, mc, hc), jnp.bfloat16),
            pltpu.VMEM((2, mc, hc), jnp.bfloat16),
            pltpu.VMEM((1, mc, hc), jnp.bfloat16),
            pltpu.VMEM((1, mc, hc), jnp.bfloat16),
            pltpu.SemaphoreType.DMA((8,)),
            pltpu.SemaphoreType.DMA((8,)),
        ],
        compiler_params=pltpu.CompilerParams(collective_id=0),
    )(x)


# device time: 49050 ns/iter; 1.7961x vs baseline; 1.0455x over previous
import jax
import jax.numpy as jnp
from jax import lax
from jax.experimental import pallas as pl
from jax.experimental.pallas import tpu as pltpu

N_DEV = 4


def kernel(x):
    m, n = x.shape
    mc = m // N_DEV
    hc = n // 2

    def body(
        x_ref,
        out_ref,
        accA,
        accB,
        rA1,
        rB1,
        rA2,
        rB2,
        send_sems,
        recv_sems,
    ):
        d = lax.axis_index("i")
        p1 = d ^ 1
        p2 = 3 - d

        barrier_sem = pltpu.get_barrier_semaphore()
        for nbr in [p1, p2]:
            pl.semaphore_signal(
                barrier_sem,
                inc=1,
                device_id=(nbr,),
                device_id_type=pl.DeviceIdType.MESH,
            )
        pl.semaphore_wait(barrier_sem, 2)

        ownA = lax.rem(d + 3, N_DEV)
        keepA = jnp.where(ownA >= 2, 2, 0)
        sendA1 = 2 - keepA
        sendA2 = lax.rem(6 - d, N_DEV)
        keepB = jnp.where(d >= 2, 2, 0)
        sendB1 = 2 - keepB
        sendB2 = p1

        def exchange(src, dst, sem_idx, partner):
            return pltpu.make_async_remote_copy(
                src_ref=src,
                dst_ref=dst,
                send_sem=send_sems.at[sem_idx],
                recv_sem=recv_sems.at[sem_idx],
                device_id=(partner,),
                device_id_type=pl.DeviceIdType.MESH,
            )

        accA[pl.ds(sendA1, 2)] = x_ref[pl.ds(sendA1 * mc, 2 * mc), 0:hc].astype(
            jnp.bfloat16
        ).reshape(2, mc, hc)
        accB[pl.ds(sendB1, 2)] = x_ref[pl.ds(sendB1 * mc, 2 * mc), hc:n].astype(
            jnp.bfloat16
        ).reshape(2, mc, hc)

        a1 = exchange(accA.at[pl.ds(sendA1, 2)], rA1, 0, p1)
        b1 = exchange(accB.at[pl.ds(sendB1, 2)], rB1, 1, p2)
        a1.start()
        b1.start()

        accA[pl.ds(keepA, 2)] = x_ref[pl.ds(keepA * mc, 2 * mc), 0:hc].astype(
            jnp.bfloat16
        ).reshape(2, mc, hc)
        accB[pl.ds(keepB, 2)] = x_ref[pl.ds(keepB * mc, 2 * mc), hc:n].astype(
            jnp.bfloat16
        ).reshape(2, mc, hc)

        a1.wait()
        accA[pl.ds(keepA, 2)] = accA[pl.ds(keepA, 2)] + rA1[...]

        a2 = exchange(accA.at[pl.ds(sendA2, 1)], rA2, 2, p2)
        a2.start()

        b1.wait()
        accB[pl.ds(keepB, 2)] = accB[pl.ds(keepB, 2)] + rB1[...]

        b2 = exchange(accB.at[pl.ds(sendB2, 1)], rB2, 3, p1)
        b2.start()

        a2.wait()
        out_ref[pl.ds(ownA * mc, mc), 0:hc] = (
            accA[pl.ds(ownA, 1)] + rA2[...]
        ).reshape(mc, hc)

        g1a = exchange(
            out_ref.at[pl.ds(ownA * mc, mc), 0:hc],
            out_ref.at[pl.ds(ownA * mc, mc), 0:hc],
            4,
            p2,
        )
        g1a.start()

        b2.wait()
        out_ref[pl.ds(d * mc, mc), hc:n] = (
            accB[pl.ds(d, 1)] + rB2[...]
        ).reshape(mc, hc)

        g1b = exchange(
            out_ref.at[pl.ds(d * mc, mc), hc:n],
            out_ref.at[pl.ds(d * mc, mc), hc:n],
            5,
            p1,
        )
        g1b.start()

        g1a.wait()
        g2a = exchange(
            out_ref.at[pl.ds(keepA * mc, 2 * mc), 0:hc],
            out_ref.at[pl.ds(keepA * mc, 2 * mc), 0:hc],
            6,
            p1,
        )
        g2a.start()

        g1b.wait()
        g2b = exchange(
            out_ref.at[pl.ds(keepB * mc, 2 * mc), hc:n],
            out_ref.at[pl.ds(keepB * mc, 2 * mc), hc:n],
            7,
            p2,
        )
        g2b.start()

        g2a.wait()
        g2b.wait()

    return pl.pallas_call(
        body,
        out_shape=jax.ShapeDtypeStruct((m, n), jnp.bfloat16),
        in_specs=[pl.BlockSpec(memory_space=pltpu.VMEM)],
        out_specs=pl.BlockSpec(memory_space=pltpu.VMEM),
        scratch_shapes=[
            pltpu.VMEM((N_DEV, mc, hc), jnp.bfloat16),
            pltpu.VMEM((N_DEV, mc, hc), jnp.bfloat16),
            pltpu.VMEM((2, mc, hc), jnp.bfloat16),
            pltpu.VMEM((2, mc, hc), jnp.bfloat16),
            pltpu.VMEM((1, mc, hc), jnp.bfloat16),
            pltpu.VMEM((1, mc, hc), jnp.bfloat16),
            pltpu.SemaphoreType.DMA((8,)),
            pltpu.SemaphoreType.DMA((8,)),
        ],
        compiler_params=pltpu.CompilerParams(collective_id=0),
    )(x)
